# baseline (speedup 1.0000x reference)
import functools

import jax
import jax.numpy as jnp
from jax import lax
from jax.experimental import pallas as pl
from jax.experimental.pallas import tpu as pltpu
from jax.experimental.pallas import tpu_sc as plsc

NC = 2
NS = 16
NW = NC * NS
L = 16
D = 64
BB = 128


def _fused_lookup(pid_t, mid_t, wpad, morph_flat, *, batch, seq):
    mesh = plsc.VectorSubcoreMesh(core_axis_name="c", subcore_axis_name="s")
    n_morph = morph_flat.shape[0]

    @functools.partial(
        pl.kernel,
        out_type=jax.ShapeDtypeStruct((seq, D, batch), jnp.float32),
        mesh=mesh,
        compiler_params=pltpu.CompilerParams(
            use_tc_tiling_on_sc=True, needs_layout_passes=False),
        scratch_types=[
            pltpu.VMEM((n_morph,), jnp.float32),
            pltpu.VMEM((BB,), jnp.int32),
            pltpu.VMEM((BB,), jnp.int32),
            pltpu.VMEM((BB,), jnp.int32),
            pltpu.VMEM((BB,), jnp.int32),
            pltpu.VMEM((BB, 2 * D), jnp.float32),
            pltpu.VMEM((BB, 2 * D), jnp.float32),
            pltpu.VMEM((D, BB), jnp.float32),
            pltpu.VMEM((D, BB), jnp.float32),
            pltpu.SemaphoreType.DMA,
            pltpu.SemaphoreType.DMA,
            pltpu.SemaphoreType.DMA,
            pltpu.SemaphoreType.DMA,
            pltpu.SemaphoreType.DMA,
            pltpu.SemaphoreType.DMA,
        ],
    )
    def body(pid_hbm, mid_hbm, wpad_hbm, morph_hbm, out_hbm,
             morph_v, idxw_a, idxm_a, idxw_b, idxm_b,
             roww_a, roww_b, obuf_a, obuf_b,
             semi_a, semi_b, semg_a, semg_b, semo_a, semo_b):
        wid = lax.axis_index("s") * NC + lax.axis_index("c")
        b0 = wid * BB

        pltpu.sync_copy(morph_hbm, morph_v)

        def idx_copies(g, idxw, idxm, semi):
            lo = g * batch + b0
            ci = pltpu.make_async_copy(pid_hbm.at[pl.ds(lo, BB)], idxw, semi)
            cj = pltpu.make_async_copy(mid_hbm.at[pl.ds(lo, BB)], idxm, semi)
            return ci, cj

        def gather_copy(roww, idxw, semg):
            return pltpu.make_async_copy(wpad_hbm.at[idxw], roww, semg)

        def out_copy(g, obuf, semo):
            return pltpu.make_async_copy(
                obuf, out_hbm.at[g, :, pl.ds(b0, BB)], semo)

        iot = jnp.arange(L, dtype=jnp.int32)

        def compute(roww, idxm, obuf):
            rots = [(iot + c) & (L - 1) for c in range(L)]

            def lg_loop(lg, carry):
                rid = iot + lg * L
                mrow = plsc.load_gather(idxm, [rid])
                m64 = mrow * D
                for ob in range(D // L):
                    for c in range(L):
                        od = rots[c] + ob * L
                        a = plsc.load_gather(roww, [rid, od])
                        b = plsc.load_gather(morph_v, [m64 + od])
                        plsc.store_scatter(obuf, [od, rid], (a + b) * 0.5)
                return carry

            lax.fori_loop(0, BB // L, lg_loop, 0, unroll=False)

        def step(g, cur, oth):
            idxw, idxm, roww, obuf, semi, semg, semo = cur
            o_idxw, o_idxm, o_roww, o_obuf, o_semi, o_semg, o_semo = oth

            gather_copy(roww, idxw, semg).wait()

            @pl.when(g + 1 < seq)
            def _():
                ci, cj = idx_copies(g + 1, o_idxw, o_idxm, o_semi)
                ci.wait()
                cj.wait()

                @pl.when(g >= 1)
                def _():
                    out_copy(g - 1, o_obuf, o_semo).wait()

                gather_copy(o_roww, o_idxw, o_semg).start()

            compute(roww, idxm, obuf)

            @pl.when(g + 2 < seq)
            def _():
                ci, cj = idx_copies(g + 2, idxw, idxm, semi)
                ci.start()
                cj.start()

            out_copy(g, obuf, semo).start()

        buf_a = (idxw_a, idxm_a, roww_a, obuf_a, semi_a, semg_a, semo_a)
        buf_b = (idxw_b, idxm_b, roww_b, obuf_b, semi_b, semg_b, semo_b)

        ci, cj = idx_copies(0, idxw_a, idxm_a, semi_a)
        ci.start()
        cj.start()
        ci, cj = idx_copies(1, idxw_b, idxm_b, semi_b)
        ci.start()
        cj.start()
        ci, cj = idx_copies(0, idxw_a, idxm_a, semi_a)
        ci.wait()
        cj.wait()
        gather_copy(roww_a, idxw_a, semg_a).start()

        def super_step(t, carry):
            step(2 * t, buf_a, buf_b)
            step(2 * t + 1, buf_b, buf_a)
            return carry

        lax.fori_loop(0, seq // 2, super_step, 0, unroll=False)
        out_copy(seq - 2, obuf_a, semo_a).wait()
        out_copy(seq - 1, obuf_b, semo_b).wait()

    return body(pid_t, mid_t, wpad, morph_flat)


_TB = 512


def _transpose_pad(word_t):
    v = word_t.shape[1]
    grid = (v + _TB - 1) // _TB

    def body(x_ref, o_ref):
        o_ref[:, :D] = x_ref[...].T

    return pl.pallas_call(
        body,
        grid=(grid,),
        in_specs=[pl.BlockSpec((D, _TB), lambda i: (0, i))],
        out_specs=pl.BlockSpec((_TB, 2 * D), lambda i: (i, 0)),
        out_shape=jax.ShapeDtypeStruct((v, 2 * D), jnp.float32),
    )(word_t)


def kernel(phrase_ids, morph_ids, word_table, morph_table):
    batch, seq = phrase_ids.shape
    pid_t = phrase_ids.T.reshape(-1)
    mid_t = morph_ids.T.reshape(-1)
    wpad = _transpose_pad(word_table.T)
    morph_flat = morph_table.reshape(-1)
    out_t = _fused_lookup(pid_t, mid_t, wpad, morph_flat,
                          batch=batch, seq=seq)
    return out_t.transpose(2, 0, 1)


# speedup vs baseline: 1.5060x; 1.5060x over previous
import functools

import jax
import jax.numpy as jnp
from jax import lax
from jax.experimental import pallas as pl
from jax.experimental.pallas import tpu as pltpu
from jax.experimental.pallas import tpu_sc as plsc

NC = 2
NS = 16
NW = NC * NS
L = 16
D = 64
BB = 128


def _fused_lookup(pid_t, mid_t, wpad, morph_flat, *, batch, seq):
    mesh = plsc.VectorSubcoreMesh(core_axis_name="c", subcore_axis_name="s")
    n_morph = morph_flat.shape[0]

    @functools.partial(
        pl.kernel,
        out_type=jax.ShapeDtypeStruct((seq, D, batch), jnp.float32),
        mesh=mesh,
        compiler_params=pltpu.CompilerParams(
            use_tc_tiling_on_sc=True, needs_layout_passes=False),
        scratch_types=[
            pltpu.VMEM((n_morph,), jnp.float32),
            pltpu.VMEM((BB,), jnp.int32),
            pltpu.VMEM((BB,), jnp.int32),
            pltpu.VMEM((BB,), jnp.int32),
            pltpu.VMEM((BB,), jnp.int32),
            pltpu.VMEM((BB, 2 * D), jnp.float32),
            pltpu.VMEM((BB, 2 * D), jnp.float32),
            pltpu.VMEM((D, BB), jnp.float32),
            pltpu.VMEM((D, BB), jnp.float32),
            pltpu.SemaphoreType.DMA,
            pltpu.SemaphoreType.DMA,
            pltpu.SemaphoreType.DMA,
            pltpu.SemaphoreType.DMA,
            pltpu.SemaphoreType.DMA,
            pltpu.SemaphoreType.DMA,
        ],
    )
    def body(pid_hbm, mid_hbm, wpad_hbm, morph_hbm, out_hbm,
             morph_v, idxw_a, idxm_a, idxw_b, idxm_b,
             roww_a, roww_b, obuf_a, obuf_b,
             semi_a, semi_b, semg_a, semg_b, semo_a, semo_b):
        wid = lax.axis_index("s") * NC + lax.axis_index("c")
        b0 = wid * BB

        pltpu.sync_copy(morph_hbm, morph_v)

        def idx_copies(g, idxw, idxm, semi):
            lo = g * batch + b0
            ci = pltpu.make_async_copy(pid_hbm.at[pl.ds(lo, BB)], idxw, semi)
            cj = pltpu.make_async_copy(mid_hbm.at[pl.ds(lo, BB)], idxm, semi)
            return ci, cj

        def gather_copy(roww, idxw, semg):
            return pltpu.make_async_copy(wpad_hbm.at[idxw], roww, semg)

        def out_copy(g, obuf, semo):
            return pltpu.make_async_copy(
                obuf, out_hbm.at[g, :, pl.ds(b0, BB)], semo)

        iot = jnp.arange(L, dtype=jnp.int32)

        def compute(roww, idxm, obuf):
            rots = [(iot + c) & (L - 1) for c in range(L)]

            def lg_loop(lg, carry):
                rid = iot + lg * L
                mrow = plsc.load_gather(idxm, [rid])
                m64 = mrow * D
                for ob in range(D // L):
                    for c in range(L):
                        od = rots[c] + ob * L
                        a = plsc.load_gather(roww, [rid, od])
                        b = plsc.load_gather(morph_v, [m64 + od])
                        plsc.store_scatter(obuf, [od, rid], (a + b) * 0.5)
                return carry

            lax.fori_loop(0, BB // L, lg_loop, 0, unroll=False)

        def step(g, cur, oth):
            idxw, idxm, roww, obuf, semi, semg, semo = cur
            o_idxw, o_idxm, o_roww, o_obuf, o_semi, o_semg, o_semo = oth

            gather_copy(roww, idxw, semg).wait()

            @pl.when(g + 1 < seq)
            def _():
                ci, cj = idx_copies(g + 1, o_idxw, o_idxm, o_semi)
                ci.wait()
                cj.wait()

                @pl.when(g >= 1)
                def _():
                    out_copy(g - 1, o_obuf, o_semo).wait()

                gather_copy(o_roww, o_idxw, o_semg).start()

            compute(roww, idxm, obuf)

            @pl.when(g + 2 < seq)
            def _():
                ci, cj = idx_copies(g + 2, idxw, idxm, semi)
                ci.start()
                cj.start()

            out_copy(g, obuf, semo).start()

        buf_a = (idxw_a, idxm_a, roww_a, obuf_a, semi_a, semg_a, semo_a)
        buf_b = (idxw_b, idxm_b, roww_b, obuf_b, semi_b, semg_b, semo_b)

        ci, cj = idx_copies(0, idxw_a, idxm_a, semi_a)
        ci.start()
        cj.start()
        ci, cj = idx_copies(1, idxw_b, idxm_b, semi_b)
        ci.start()
        cj.start()
        ci, cj = idx_copies(0, idxw_a, idxm_a, semi_a)
        ci.wait()
        cj.wait()
        gather_copy(roww_a, idxw_a, semg_a).start()

        def super_step(t, carry):
            step(2 * t, buf_a, buf_b)
            step(2 * t + 1, buf_b, buf_a)
            return carry

        lax.fori_loop(0, seq // 2, super_step, 0, unroll=False)
        out_copy(seq - 2, obuf_a, semo_a).wait()
        out_copy(seq - 1, obuf_b, semo_b).wait()

    return body(pid_t, mid_t, wpad, morph_flat)


def kernel(phrase_ids, morph_ids, word_table, morph_table):
    batch, seq = phrase_ids.shape
    pid_t = phrase_ids.T.reshape(-1)
    mid_t = morph_ids.T.reshape(-1)
    wpad = jnp.pad(word_table, ((0, 0), (0, 2 * D - word_table.shape[1])))
    morph_flat = morph_table.reshape(-1)
    out_t = _fused_lookup(pid_t, mid_t, wpad, morph_flat,
                          batch=batch, seq=seq)
    return out_t.transpose(2, 0, 1)
